# baseline (speedup 1.0000x reference)
import functools

import jax
import jax.numpy as jnp
from jax import lax
from jax.experimental import pallas as pl
from jax.experimental.pallas import tpu as pltpu
from jax.experimental.pallas import tpu_sc as plsc

_N_HEADS = 32
_MAX_CTX = 8192
_HDIM = 128
_QLEN = 16


def _sc_body(ko_hbm, vo_hbm, pos_hbm, k_hbm, v_hbm,
             idx_v, src_v, krows_v, vrows_v, sem_k, sem_v):
    wid = lax.axis_index("s") * 2 + lax.axis_index("c")
    row0 = wid * _MAX_CTX
    pltpu.sync_copy(pos_hbm, idx_v)
    pvec = idx_v[...]
    m = lax.iota(jnp.int32, _QLEN)
    for j in range(_QLEN):
        bj = pvec.at[jnp.full((_QLEN,), j, jnp.int32)].get(
            mode="promise_in_bounds")
        m = jnp.where(pvec == bj, jnp.maximum(m, j), m)
    src_v[...] = m + wid * _QLEN
    idx_v[...] = pvec + row0
    gk = pltpu.make_async_copy(k_hbm.at[src_v], krows_v, sem_k)
    gv = pltpu.make_async_copy(v_hbm.at[src_v], vrows_v, sem_v)
    gk.start()
    gv.start()
    gk.wait()
    gv.wait()
    sk = pltpu.make_async_copy(krows_v, ko_hbm.at[idx_v], sem_k)
    sv = pltpu.make_async_copy(vrows_v, vo_hbm.at[idx_v], sem_v)
    sk.start()
    sv.start()
    sk.wait()
    sv.wait()


_sc_scatter = functools.partial(
    pl.kernel,
    mesh=plsc.VectorSubcoreMesh(core_axis_name="c", subcore_axis_name="s"),
    scratch_types=[
        pltpu.VMEM((_QLEN,), jnp.int32),
        pltpu.VMEM((_QLEN,), jnp.int32),
        pltpu.VMEM((_QLEN, _HDIM), jnp.float32),
        pltpu.VMEM((_QLEN, _HDIM), jnp.float32),
        pltpu.SemaphoreType.DMA,
        pltpu.SemaphoreType.DMA,
    ],
)(_sc_body)


def kernel(k_cache, v_cache, pos_ids, k, v):
    ko_ref = jax.new_ref(k_cache.reshape(_N_HEADS * _MAX_CTX, _HDIM))
    vo_ref = jax.new_ref(v_cache.reshape(_N_HEADS * _MAX_CTX, _HDIM))
    _sc_scatter(ko_ref, vo_ref,
                pos_ids.astype(jnp.int32),
                k.reshape(_N_HEADS * _QLEN, _HDIM),
                v.reshape(_N_HEADS * _QLEN, _HDIM))
    return (ko_ref[...].reshape(k_cache.shape),
            vo_ref[...].reshape(v_cache.shape))
